# Initial kernel scaffold; baseline (speedup 1.0000x reference)
import jax
import jax.numpy as jnp
from jax.experimental import pallas as pl


def kernel(x, edge_index, edge_attr, Wq, bq, Wk, Wv, Wskip, bskip, Wbeta, bbeta):
    raise NotImplementedError("write your pallas kernel here")


# speedup vs baseline: 18.4695x; 18.4695x over previous
import functools

import numpy as np
import jax
import jax.numpy as jnp
from jax import lax
from jax.experimental import pallas as pl
from jax.experimental.pallas import tpu as pltpu
from jax.experimental.pallas import tpu_sc as plsc

N = 10000
E = 320000
D_IN = 128
H = 4
D_H = 32
HD = H * D_H
COLS = HD + 16
SCALE = 1.0 / np.sqrt(D_H)

NC = 2
NS = 16
NW = NC * NS
EPT = E // NW
C = 40
NCHUNK = EPT // C
NPAD = 10240
RPT = NPAD // NS
DRPT = NPAD // 8 // NS


def _qkvs_body(x_ref, w_ref, bq_ref, bs_ref, q_ref, kv_ref, xr_ref):
    y = jnp.dot(x_ref[...], w_ref[...], preferred_element_type=jnp.float32)
    q_ref[...] = y[:, :HD] + bq_ref[...]
    kv_ref[...] = y[:, HD:3 * HD]
    xr_ref[...] = y[:, 3 * HD:] + bs_ref[...]


def _qkvs(x, wcat, bq, bs):
    bn = 2000
    grid = (N // bn,)
    return pl.pallas_call(
        _qkvs_body,
        grid=grid,
        in_specs=[
            pl.BlockSpec((bn, D_IN), lambda i: (i, 0)),
            pl.BlockSpec((D_IN, 4 * HD), lambda i: (0, 0)),
            pl.BlockSpec((1, HD), lambda i: (0, 0)),
            pl.BlockSpec((1, HD), lambda i: (0, 0)),
        ],
        out_specs=[
            pl.BlockSpec((bn, HD), lambda i: (i, 0)),
            pl.BlockSpec((bn, 2 * HD), lambda i: (i, 0)),
            pl.BlockSpec((bn, HD), lambda i: (i, 0)),
        ],
        out_shape=[
            jax.ShapeDtypeStruct((N, HD), jnp.float32),
            jax.ShapeDtypeStruct((N, 2 * HD), jnp.float32),
            jax.ShapeDtypeStruct((N, HD), jnp.float32),
        ],
    )(x, wcat, bq, bs)


def _sc_body(q_hbm, kv_hbm, src_hbm, dst_hbm, ea_hbm, z_hbm,
             num_hbm, den_hbm,
             sidx, didx, didx8, qrows, kvrows, earows, msg, dmsg, dstage,
             acc_num, acc_den, sem):
    c = lax.axis_index("c")
    s = lax.axis_index("s")
    w = c * NS + s

    pltpu.sync_copy(z_hbm.at[pl.ds(s * RPT, RPT)], acc_num.at[pl.ds(s * RPT, RPT)])
    pltpu.sync_copy(z_hbm.at[pl.ds(s * DRPT, DRPT)], acc_den.at[pl.ds(s * DRPT, DRPT)])
    pltpu.sync_copy(z_hbm.at[pl.ds(0, C)], dmsg)
    plsc.subcore_barrier()

    lanes = lax.iota(jnp.int32, 16)
    zvec = jnp.zeros((16,), jnp.float32)
    bfly = [lanes ^ sh for sh in (8, 4, 2, 1)]
    gdn = lax.GatherDimensionNumbers(
        offset_dims=(), collapsed_slice_dims=(0,), start_index_map=(0,))

    def xlane_sum(t):
        for idx in bfly:
            t = t + lax.gather(t, idx[:, None], gdn, slice_sizes=(1,),
                               mode=lax.GatherScatterMode.PROMISE_IN_BOUNDS)
        return t

    gbases = list(range(0, C - 15, 16))
    if C % 16:
        gbases.append(C - 16)

    def chunk_body(j, carry):
        base = w * EPT + j * C
        pltpu.sync_copy(src_hbm.at[pl.ds(base, C)], sidx)
        pltpu.sync_copy(dst_hbm.at[pl.ds(base, C)], didx)
        cp_q = pltpu.async_copy(q_hbm.at[didx], qrows, sem)
        cp_kv = pltpu.async_copy(kv_hbm.at[sidx], kvrows, sem)
        pltpu.sync_copy(ea_hbm.at[pl.ds(base, C)], earows)
        for b in gbases:
            didx8[pl.ds(b, 16)] = lax.shift_right_logical(didx[pl.ds(b, 16)], 3)
        cp_q.wait()
        cp_kv.wait()

        def edge_body(i, carry2):
            den = zvec
            for h in range(H):
                o1 = h * D_H
                o2 = o1 + 16
                eaa = earows[i, pl.ds(o1, 16)]
                eab = earows[i, pl.ds(o2, 16)]
                ka = kvrows[i, pl.ds(o1, 16)] + eaa
                kb = kvrows[i, pl.ds(o2, 16)] + eab
                t = qrows[i, pl.ds(o1, 16)] * ka + qrows[i, pl.ds(o2, 16)] * kb
                p = jnp.exp(xlane_sum(t) * SCALE)
                msg[i, pl.ds(o1, 16)] = p * (kvrows[i, pl.ds(HD + o1, 16)] + eaa)
                msg[i, pl.ds(o2, 16)] = p * (kvrows[i, pl.ds(HD + o2, 16)] + eab)
                den = den + jnp.where(lanes == h, p, 0.0)
            dstage[i, :] = den
            return carry2

        lax.fori_loop(0, C, edge_body, 0)

        for b in gbases:
            rows = b + lanes
            colbase = (didx[pl.ds(b, 16)] & 7) * 16
            for h in range(H):
                vals = plsc.load_gather(dstage, [rows, jnp.broadcast_to(h, (16,))])
                plsc.store_scatter(dmsg, [rows, colbase + h], vals)

        pltpu.sync_copy(msg, acc_num.at[didx], add=True)
        pltpu.sync_copy(dmsg, acc_den.at[didx8], add=True)

        for b in gbases:
            rows = b + lanes
            colbase = (didx[pl.ds(b, 16)] & 7) * 16
            for h in range(H):
                plsc.store_scatter(dmsg, [rows, colbase + h], zvec)
        return carry

    lax.fori_loop(0, NCHUNK, chunk_body, 0)
    plsc.subcore_barrier()
    pltpu.sync_copy(acc_num.at[pl.ds(s * RPT, RPT)],
                    num_hbm.at[c, pl.ds(s * RPT, RPT)])
    pltpu.sync_copy(acc_den.at[pl.ds(s * DRPT, DRPT)],
                    den_hbm.at[c, pl.ds(s * DRPT, DRPT)])


def _sc_edge(q, kv, src, dst, ea, zeros):
    mesh = plsc.VectorSubcoreMesh(core_axis_name="c", subcore_axis_name="s")
    f = functools.partial(
        pl.kernel,
        mesh=mesh,
        compiler_params=pltpu.CompilerParams(needs_layout_passes=False),
        out_type=(jax.ShapeDtypeStruct((NC, NPAD, HD), jnp.float32),
                  jax.ShapeDtypeStruct((NC, NPAD // 8, HD), jnp.float32)),
        scratch_types=[
            pltpu.VMEM((C,), jnp.int32),
            pltpu.VMEM((C,), jnp.int32),
            pltpu.VMEM((C,), jnp.int32),
            pltpu.VMEM((C, HD), jnp.float32),
            pltpu.VMEM((C, 2 * HD), jnp.float32),
            pltpu.VMEM((C, HD), jnp.float32),
            pltpu.VMEM((C, HD), jnp.float32),
            pltpu.VMEM((C, HD), jnp.float32),
            pltpu.VMEM((C, 16), jnp.float32),
            pltpu.VMEM_SHARED((NPAD, HD), jnp.float32),
            pltpu.VMEM_SHARED((NPAD // 8, HD), jnp.float32),
            pltpu.SemaphoreType.DMA,
        ],
    )(_sc_body)
    return f(q, kv, src, dst, ea, zeros)


def _final_body(num_ref, d16_ref, xr_ref, wo_ref, wx_ref, bb_ref, s_ref, o_ref):
    num = num_ref[0] + num_ref[1]
    den16 = d16_ref[0] + d16_ref[1]
    den = jnp.dot(den16, s_ref[...], preferred_element_type=jnp.float32)
    outm = num / (den + 1e-16)
    xr = xr_ref[...]
    z = (jnp.dot(outm, wo_ref[...], preferred_element_type=jnp.float32)
         + jnp.dot(xr, wx_ref[...], preferred_element_type=jnp.float32)
         + bb_ref[...])
    beta = jax.nn.sigmoid(z)
    o_ref[...] = beta * xr + (1.0 - beta) * outm


def _final(num, d16, xr, wo, wx, bb, smat):
    bn = 2000
    grid = (N // bn,)
    return pl.pallas_call(
        _final_body,
        grid=grid,
        in_specs=[
            pl.BlockSpec((NC, bn, HD), lambda i: (0, i, 0)),
            pl.BlockSpec((NC, bn, 16), lambda i: (0, i, 0)),
            pl.BlockSpec((bn, HD), lambda i: (i, 0)),
            pl.BlockSpec((HD, HD), lambda i: (0, 0)),
            pl.BlockSpec((HD, HD), lambda i: (0, 0)),
            pl.BlockSpec((1, HD), lambda i: (0, 0)),
            pl.BlockSpec((16, HD), lambda i: (0, 0)),
        ],
        out_specs=pl.BlockSpec((bn, HD), lambda i: (i, 0)),
        out_shape=jax.ShapeDtypeStruct((N, HD), jnp.float32),
    )(num, d16, xr, wo, wx, bb, smat)


_SMAT = np.zeros((16, HD), np.float32)
for _h in range(H):
    _SMAT[_h, _h * D_H:(_h + 1) * D_H] = 1.0


def kernel(x, edge_index, edge_attr, Wq, bq, Wk, Wv, Wskip, bskip, Wbeta, bbeta):
    src = edge_index[0]
    dst = edge_index[1]
    wcat = jnp.concatenate([Wq, Wk, Wv, Wskip], axis=1)
    q, kv, xr = _qkvs(x, wcat, bq[None, :], bskip[None, :])
    zeros = jnp.zeros((NPAD, HD), jnp.float32)
    nd_num, nd_den = _sc_edge(q, kv, src, dst, edge_attr, zeros)
    num = nd_num[:, :N]
    d16 = nd_den.reshape(NC, NPAD, 16)[:, :N]
    wo = Wbeta[:HD] + Wbeta[2 * HD:]
    wx = Wbeta[HD:2 * HD] - Wbeta[2 * HD:]
    return _final(num, d16, xr, wo, wx, bbeta[None, :], jnp.asarray(_SMAT))
